# Initial kernel scaffold; baseline (speedup 1.0000x reference)
import functools

import jax
import jax.numpy as jnp
from jax import lax
from jax.experimental import pallas as pl
from jax.experimental.pallas import tpu as pltpu
from jax.experimental.pallas import tpu_sc as plsc

_L = 16
_D = 512
_NC = _D // _L
_B = 8
_M = 8
_R = _B * _M

_PAIRS = [(i, j) for i in range(_B) for j in range(i + 1, _B)]
_COS_MARGIN = 0.2


def _rsqrt_s(x):
    xc = jnp.maximum(x, jnp.float32(1e-24))
    i = lax.bitcast_convert_type(xc, jnp.int32)
    i = jnp.int32(0x5F3759DF) - jnp.right_shift(i, 1)
    y = lax.bitcast_convert_type(i, jnp.float32)
    for _ in range(4):
        y = y * (jnp.float32(1.5) - jnp.float32(0.5) * xc * y * y)
    return y


def _make_kernel():
    mesh = plsc.VectorSubcoreMesh(core_axis_name="c", subcore_axis_name="s")

    @functools.partial(
        pl.kernel,
        out_type=jax.ShapeDtypeStruct((_L,), jnp.float32),
        mesh=mesh,
        scratch_types=[
            pltpu.VMEM((_R,), jnp.int32),
            pltpu.VMEM((_R, _D), jnp.float32),
            pltpu.VMEM((_B * _D,), jnp.float32),
            pltpu.VMEM((_L,), jnp.float32),
            pltpu.SemaphoreType.DMA,
        ],
    )
    def _k(emb_hbm, idx_hbm, out_hbm, idx_v, rows_v, cent_v, res_v, sem):
        cid = lax.axis_index("c")
        sid = lax.axis_index("s")

        @pl.when(jnp.logical_and(cid == 0, sid == 0))
        def _():
            pltpu.sync_copy(idx_hbm, idx_v)
            pltpu.async_copy(emb_hbm.at[idx_v], rows_v, sem).wait()

            def zero_body(k, carry):
                cent_v[pl.ds(k * _L, _L)] = jnp.zeros((_L,), jnp.float32)
                return carry
            lax.fori_loop(0, _B * _NC, zero_body, 0)

            def row_body(r, carry):
                def sq_body(c, a):
                    v = rows_v[r, pl.ds(c * _L, _L)]
                    return a + v * v
                ss = lax.fori_loop(0, _NC, sq_body,
                                   jnp.zeros((_L,), jnp.float32))
                inv = _rsqrt_s(jnp.maximum(jnp.sum(ss), jnp.float32(1e-16)))
                b = r // _M

                def add_body(c, carry2):
                    off = b * _D + c * _L
                    v = rows_v[r, pl.ds(c * _L, _L)]
                    cent_v[pl.ds(off, _L)] = cent_v[pl.ds(off, _L)] + v * inv
                    return carry2
                lax.fori_loop(0, _NC, add_body, 0)
                return carry
            lax.fori_loop(0, _R, row_body, 0)

            invs = []
            nsum = jnp.float32(0.0)
            for b in range(_B):
                def bsq_body(c, a, b=b):
                    v = cent_v[pl.ds(b * _D + c * _L, _L)]
                    return a + v * v
                ss = lax.fori_loop(0, _NC, bsq_body,
                                   jnp.zeros((_L,), jnp.float32))
                tot = jnp.maximum(jnp.sum(ss), jnp.float32(1e-24))
                invb = _rsqrt_s(tot)
                invs.append(invb)
                nsum = nsum + tot * invb
            cohesion = jnp.float32(1.0) - nsum * jnp.float32(1.0 / 64.0)

            zeros28 = tuple(jnp.zeros((_L,), jnp.float32) for _ in _PAIRS)

            def pair_body(c, accs):
                vs = [cent_v[pl.ds(b * _D + c * _L, _L)] for b in range(_B)]
                return tuple(acc + vs[i] * vs[j]
                             for acc, (i, j) in zip(accs, _PAIRS))
            accs = lax.fori_loop(0, _NC, pair_body, zeros28)

            sep = jnp.float32(0.0)
            for acc, (i, j) in zip(accs, _PAIRS):
                cos = jnp.sum(acc) * invs[i] * invs[j]
                sep = sep + jnp.maximum(cos - jnp.float32(_COS_MARGIN),
                                        jnp.float32(0.0))
            sep = sep * jnp.float32(1.0 / len(_PAIRS))

            loss = cohesion + sep
            res_v[...] = jnp.broadcast_to(loss, (_L,))
            pltpu.sync_copy(res_v, out_hbm)

    return _k


_sc_loss = _make_kernel()


def kernel(embeddings, branch_members):
    idx = branch_members.reshape(-1).astype(jnp.int32)
    out = _sc_loss(embeddings, idx)
    return out[0]


# speedup vs baseline: 4.1176x; 4.1176x over previous
import functools

import jax
import jax.numpy as jnp
from jax import lax
from jax.experimental import pallas as pl
from jax.experimental.pallas import tpu as pltpu
from jax.experimental.pallas import tpu_sc as plsc

_L = 16
_D = 512
_NC = _D // _L
_B = 8
_M = 8
_R = _B * _M

_PAIRS = [(i, j) for i in range(_B) for j in range(i + 1, _B)]
_COS_MARGIN = 0.2


def _lane_sum(v):
    dnums = lax.GatherDimensionNumbers(
        offset_dims=(), collapsed_slice_dims=(0,), start_index_map=(0,))
    iota = lax.iota(jnp.int32, _L)
    for s in (8, 4, 2, 1):
        idx = jnp.bitwise_xor(iota, jnp.int32(s))
        perm = lax.gather(v, idx[:, None], dnums, slice_sizes=(1,),
                          mode=lax.GatherScatterMode.PROMISE_IN_BOUNDS)
        v = v + perm
    return v


def _rsqrt_v(x):
    xc = jnp.maximum(x, jnp.float32(1e-24))
    i = lax.bitcast_convert_type(xc, jnp.int32)
    i = jnp.int32(0x5F3759DF) - jnp.right_shift(i, 1)
    y = lax.bitcast_convert_type(i, jnp.float32)
    for _ in range(4):
        y = y * (jnp.float32(1.5) - jnp.float32(0.5) * xc * y * y)
    return y


def _make_kernel():
    mesh = plsc.VectorSubcoreMesh(core_axis_name="c", subcore_axis_name="s")

    @functools.partial(
        pl.kernel,
        out_type=jax.ShapeDtypeStruct((_L,), jnp.float32),
        mesh=mesh,
        scratch_types=[
            pltpu.VMEM((_R,), jnp.int32),
            pltpu.VMEM((_R, _D), jnp.float32),
            pltpu.VMEM((_B * _D,), jnp.float32),
            pltpu.VMEM((_L,), jnp.float32),
            pltpu.SemaphoreType.DMA,
        ],
    )
    def _k(emb_hbm, idx_hbm, out_hbm, idx_v, rows_v, cent_v, res_v, sem):
        cid = lax.axis_index("c")
        sid = lax.axis_index("s")

        @pl.when(jnp.logical_and(cid == 0, sid == 0))
        def _():
            pltpu.sync_copy(idx_hbm, idx_v)
            pltpu.async_copy(emb_hbm.at[idx_v], rows_v, sem).wait()

            def zero_body(k, carry):
                cent_v[pl.ds(k * _L, _L)] = jnp.zeros((_L,), jnp.float32)
                return carry
            lax.fori_loop(0, _B * _NC, zero_body, 0)

            def row_body(r, carry):
                def sq_body(c, a):
                    v = rows_v[r, pl.ds(c * _L, _L)]
                    return a + v * v
                ss = lax.fori_loop(0, _NC, sq_body,
                                   jnp.zeros((_L,), jnp.float32))
                inv = _rsqrt_v(jnp.maximum(_lane_sum(ss), jnp.float32(1e-16)))
                b = r // _M

                def add_body(c, carry2):
                    off = b * _D + c * _L
                    v = rows_v[r, pl.ds(c * _L, _L)]
                    cent_v[pl.ds(off, _L)] = cent_v[pl.ds(off, _L)] + v * inv
                    return carry2
                lax.fori_loop(0, _NC, add_body, 0)
                return carry
            lax.fori_loop(0, _R, row_body, 0)

            invs = []
            nsum = jnp.zeros((_L,), jnp.float32)
            for b in range(_B):
                def bsq_body(c, a, b=b):
                    v = cent_v[pl.ds(b * _D + c * _L, _L)]
                    return a + v * v
                ss = lax.fori_loop(0, _NC, bsq_body,
                                   jnp.zeros((_L,), jnp.float32))
                tot = jnp.maximum(_lane_sum(ss), jnp.float32(1e-24))
                invb = _rsqrt_v(tot)
                invs.append(invb)
                nsum = nsum + tot * invb
            cohesion = jnp.float32(1.0) - nsum * jnp.float32(1.0 / 64.0)

            zeros28 = tuple(jnp.zeros((_L,), jnp.float32) for _ in _PAIRS)

            def pair_body(c, accs):
                vs = [cent_v[pl.ds(b * _D + c * _L, _L)] for b in range(_B)]
                return tuple(acc + vs[i] * vs[j]
                             for acc, (i, j) in zip(accs, _PAIRS))
            accs = lax.fori_loop(0, _NC, pair_body, zeros28)

            sep = jnp.zeros((_L,), jnp.float32)
            for acc, (i, j) in zip(accs, _PAIRS):
                cos = _lane_sum(acc) * invs[i] * invs[j]
                sep = sep + jnp.maximum(cos - jnp.float32(_COS_MARGIN),
                                        jnp.float32(0.0))
            sep = sep * jnp.float32(1.0 / len(_PAIRS))

            loss = cohesion + sep
            res_v[...] = loss
            pltpu.sync_copy(res_v, out_hbm)

    return _k


_sc_loss = _make_kernel()


def kernel(embeddings, branch_members):
    idx = branch_members.reshape(-1).astype(jnp.int32)
    out = _sc_loss(embeddings, idx)
    return out[0]
